# Initial kernel scaffold; baseline (speedup 1.0000x reference)
import jax
import jax.numpy as jnp
from jax.experimental import pallas as pl


def kernel(x, edge_index, W1, b1, W2, b2, Wp1, bp1, Wp2, bp2):
    raise NotImplementedError("write your pallas kernel here")


# speedup vs baseline: 6.3322x; 6.3322x over previous
import functools

import jax
import jax.numpy as jnp
from jax import lax
from jax.experimental import pallas as pl
from jax.experimental.pallas import tpu as pltpu
from jax.experimental.pallas import tpu_sc as plsc

N = 10000
E = 160000
D_IN = 256
D_HID = 512

NP_ = 10240
EP = 163840
NC = 2
NS = 16
ROWS_PER_TILE = NP_ // NS
RB = 1024
GRID_R = NP_ // RB


def _sc_degree(dstp):
    ept = EP // (NC * NS)
    nbatch = ept // 128

    @functools.partial(
        pl.kernel,
        out_type=jax.ShapeDtypeStruct((NC * NP_, 16), jnp.float32),
        mesh=plsc.VectorSubcoreMesh(core_axis_name="c", subcore_axis_name="s"),
        scratch_types=[
            pltpu.VMEM_SHARED((NP_, 16), jnp.float32),
            pltpu.VMEM((128,), jnp.int32),
            pltpu.VMEM((128, 16), jnp.float32),
            pltpu.VMEM((128, 16), jnp.float32),
        ],
    )
    def k(dst_hbm, out_hbm, acc_sh, idx_v, ones_v, zb_v):
        c = lax.axis_index("c")
        s = lax.axis_index("s")
        wid = s * NC + c
        zero16 = jnp.zeros((16,), jnp.float32)
        ones16 = jnp.ones((16,), jnp.float32)

        def init_loop(i, carry):
            ones_v[i, :] = ones16
            zb_v[i, :] = zero16
            return carry

        lax.fori_loop(0, 128, init_loop, 0)
        for kk in range(ROWS_PER_TILE // 128):
            pltpu.sync_copy(
                zb_v, acc_sh.at[pl.ds(s * ROWS_PER_TILE + kk * 128, 128)])
        plsc.subcore_barrier()

        base = wid * ept

        def body(b, carry):
            pltpu.sync_copy(dst_hbm.at[pl.ds(base + b * 128, 128)], idx_v)
            pltpu.sync_copy(ones_v, acc_sh.at[idx_v], add=True)
            return carry

        lax.fori_loop(0, nbatch, body, 0)
        plsc.subcore_barrier()
        for kk in range(ROWS_PER_TILE // 128):
            r0 = s * ROWS_PER_TILE + kk * 128
            pltpu.sync_copy(acc_sh.at[pl.ds(r0, 128)],
                            out_hbm.at[pl.ds(c * NP_ + r0, 128)])

    return k(dstp)


def _sc_aggregate(g_tab, srcp, dstp, nb):
    ept = EP // NS
    nbatch = ept // 128
    passes = nb // NC

    @functools.partial(
        pl.kernel,
        out_type=jax.ShapeDtypeStruct((nb * NP_, 128), jnp.float32),
        mesh=plsc.VectorSubcoreMesh(core_axis_name="c", subcore_axis_name="s"),
        scratch_types=[
            pltpu.VMEM_SHARED((NP_, 128), jnp.float32),
            pltpu.VMEM((128,), jnp.int32),
            pltpu.VMEM((128,), jnp.int32),
            pltpu.VMEM((128, 128), jnp.float32),
            pltpu.VMEM((128, 128), jnp.float32),
            pltpu.SemaphoreType.DMA,
        ],
    )
    def k(g_hbm, src_hbm, dst_hbm, out_hbm, acc_sh, idxs_v, idxd_v, rows_v,
          zb_v, sem):
        c = lax.axis_index("c")
        s = lax.axis_index("s")
        zero16 = jnp.zeros((16,), jnp.float32)

        def zb_loop(i, carry):
            for g in range(8):
                zb_v[i, pl.ds(g * 16, 16)] = zero16
            return carry

        lax.fori_loop(0, 128, zb_loop, 0)

        def zero_acc_slice():
            for kk in range(ROWS_PER_TILE // 128):
                pltpu.sync_copy(
                    zb_v, acc_sh.at[pl.ds(s * ROWS_PER_TILE + kk * 128, 128)])

        zero_acc_slice()
        plsc.subcore_barrier()

        for p in range(passes):
            blk = c * passes + p
            row_off = blk * NP_

            def body(b, carry):
                ebase = s * ept + b * 128
                pltpu.sync_copy(src_hbm.at[pl.ds(ebase, 128)], idxs_v)
                pltpu.sync_copy(dst_hbm.at[pl.ds(ebase, 128)], idxd_v)
                for g in range(8):
                    idxs_v[pl.ds(g * 16, 16)] = idxs_v[pl.ds(g * 16, 16)] + row_off
                pltpu.async_copy(g_hbm.at[idxs_v], rows_v, sem).wait()
                pltpu.sync_copy(rows_v, acc_sh.at[idxd_v], add=True)
                return carry

            lax.fori_loop(0, nbatch, body, 0)
            plsc.subcore_barrier()
            for kk in range(ROWS_PER_TILE // 128):
                r0 = s * ROWS_PER_TILE + kk * 128
                pltpu.sync_copy(acc_sh.at[pl.ds(r0, 128)],
                                out_hbm.at[pl.ds(row_off + r0, 128)])
            if p + 1 < passes:
                zero_acc_slice()
                plsc.subcore_barrier()

    return k(g_tab, srcp, dstp)


def _tc_prescale(xp, hist):
    nb = D_IN // 128

    def body(x_ref, hist_ref, g0_ref, dinv_ref):
        h = hist_ref[...]
        deg = 1.0 + h[0, :, 0] + h[1, :, 0]
        dinv = lax.rsqrt(deg)[:, None]
        g0_ref[...] = x_ref[...] * dinv
        dinv_ref[...] = dinv

    return pl.pallas_call(
        body,
        grid=(nb, GRID_R),
        in_specs=[
            pl.BlockSpec((RB, 128), lambda j, i: (i, j)),
            pl.BlockSpec((NC, RB, 16), lambda j, i: (0, i, 0)),
        ],
        out_specs=[
            pl.BlockSpec((RB, 128), lambda j, i: (j * GRID_R + i, 0)),
            pl.BlockSpec((RB, 1), lambda j, i: (i, 0)),
        ],
        out_shape=[
            jax.ShapeDtypeStruct((nb * NP_, 128), jnp.float32),
            jax.ShapeDtypeStruct((NP_, 1), jnp.float32),
        ],
    )(xp, hist.reshape(NC, NP_, 16))


def _tc_layer1(acc0, xp, dinv, W1, b1):
    nb_out = D_HID // 128

    def body(acc_ref, x_ref, dinv_ref, w_ref, b_ref, h1_ref, g1_ref):
        cat = jnp.concatenate([acc_ref[0], acc_ref[1]], axis=1)
        dv = dinv_ref[...]
        s0 = dv * cat + (dv * dv) * x_ref[...]
        h = lax.dot_general(s0, w_ref[...], (((1,), (1,)), ((), ())),
                            preferred_element_type=jnp.float32)
        h = jnp.maximum(h + b_ref[...], 0.0)
        h1_ref[...] = h
        g = dv * h
        for j in range(nb_out):
            g1_ref[j] = g[:, j * 128:(j + 1) * 128]

    return pl.pallas_call(
        body,
        grid=(GRID_R,),
        in_specs=[
            pl.BlockSpec((2, RB, 128), lambda i: (0, i, 0)),
            pl.BlockSpec((RB, D_IN), lambda i: (i, 0)),
            pl.BlockSpec((RB, 1), lambda i: (i, 0)),
            pl.BlockSpec((D_HID, D_IN), lambda i: (0, 0)),
            pl.BlockSpec((1, D_HID), lambda i: (0, 0)),
        ],
        out_specs=[
            pl.BlockSpec((RB, D_HID), lambda i: (i, 0)),
            pl.BlockSpec((nb_out, RB, 128), lambda i: (0, i, 0)),
        ],
        out_shape=[
            jax.ShapeDtypeStruct((NP_, D_HID), jnp.float32),
            jax.ShapeDtypeStruct((nb_out, NP_, 128), jnp.float32),
        ],
    )(acc0.reshape(2, NP_, 128), xp, dinv, W1, b1.reshape(1, D_HID))


def _tc_head(acc1, h1, dinv, W2, b2, Wp1a, Wp1b, bp1, Wp2, bp2):
    def body(acc_ref, h1_ref, dinv_ref, w2_ref, b2_ref, wa_ref, wb_ref,
             bp1_ref, wp2_ref, bp2_ref, out_ref):
        cat = jnp.concatenate([acc_ref[j] for j in range(4)], axis=1)
        dv = dinv_ref[...]
        h1b = h1_ref[...]
        s1 = dv * cat + (dv * dv) * h1b
        h2 = lax.dot_general(s1, w2_ref[...], (((1,), (1,)), ((), ())),
                             preferred_element_type=jnp.float32)
        h2 = jnp.maximum(h2 + b2_ref[...], 0.0)
        p = (lax.dot_general(h1b, wa_ref[...], (((1,), (1,)), ((), ())),
                             preferred_element_type=jnp.float32)
             + lax.dot_general(h2, wb_ref[...], (((1,), (1,)), ((), ())),
                               preferred_element_type=jnp.float32))
        p = jnp.maximum(p + bp1_ref[...], 0.0)
        pred = jnp.sum(p * wp2_ref[...], axis=1, keepdims=True)
        out_ref[...] = pred + bp2_ref[0, 0]

    return pl.pallas_call(
        body,
        grid=(GRID_R,),
        in_specs=[
            pl.BlockSpec((4, RB, 128), lambda i: (0, i, 0)),
            pl.BlockSpec((RB, D_HID), lambda i: (i, 0)),
            pl.BlockSpec((RB, 1), lambda i: (i, 0)),
            pl.BlockSpec((D_HID, D_HID), lambda i: (0, 0)),
            pl.BlockSpec((1, D_HID), lambda i: (0, 0)),
            pl.BlockSpec((128, D_HID), lambda i: (0, 0)),
            pl.BlockSpec((128, D_HID), lambda i: (0, 0)),
            pl.BlockSpec((1, 128), lambda i: (0, 0)),
            pl.BlockSpec((1, 128), lambda i: (0, 0)),
            pl.BlockSpec((1, 1), lambda i: (0, 0)),
        ],
        out_specs=pl.BlockSpec((RB, 1), lambda i: (i, 0)),
        out_shape=jax.ShapeDtypeStruct((NP_, 1), jnp.float32),
    )(acc1.reshape(4, NP_, 128), h1, dinv, W2, b2.reshape(1, D_HID),
      Wp1a, Wp1b, bp1.reshape(1, 128), Wp2, bp2.reshape(1, 1))


@jax.jit
def kernel(x, edge_index, W1, b1, W2, b2, Wp1, bp1, Wp2, bp2):
    pad = EP - E
    srcp = jnp.concatenate(
        [edge_index[0], jnp.full((pad,), N, jnp.int32)])
    dstp = jnp.concatenate(
        [edge_index[1], jnp.full((pad,), N, jnp.int32)])
    xp = jnp.pad(x, ((0, NP_ - N), (0, 0)))

    hist = _sc_degree(dstp)
    g0, dinv = _tc_prescale(xp, hist)
    acc0 = _sc_aggregate(g0, srcp, dstp, 2)
    h1, g1 = _tc_layer1(acc0, xp, dinv, W1, b1)
    acc1 = _sc_aggregate(g1.reshape(-1, 128), srcp, dstp, 4)
    pred = _tc_head(acc1, h1, dinv, W2, b2, Wp1[:, :D_HID], Wp1[:, D_HID:],
                    bp1, Wp2, bp2)
    return pred[:N]
